# baseline (speedup 1.0000x reference)
import functools

import jax
import jax.numpy as jnp
from jax import lax
from jax.experimental import pallas as pl
from jax.experimental.pallas import tpu as pltpu
from jax.experimental.pallas import tpu_sc as plsc

BATCH = 16384
FIELDS = 26
EMB_DIM = 32
LANES = 16

NUM_CORES = 2
NUM_SUBCORES = 16
NUM_WORKERS = NUM_CORES * NUM_SUBCORES
NB = BATCH // NUM_WORKERS
NG = 256

TABLE_ROWS = 1000000
PACK = 128 // EMB_DIM
T2_ROWS = TABLE_ROWS // PACK
CH = 320
NCHUNK = TABLE_ROWS // CH


@jax.jit
def _sc_embed(d, table):
    mesh = plsc.VectorSubcoreMesh(core_axis_name="c", subcore_axis_name="s")

    @functools.partial(
        pl.kernel,
        mesh=mesh,
        out_type=jax.ShapeDtypeStruct((T2_ROWS, 128), jnp.float32),
        scratch_types=[
            pltpu.VMEM((CH, EMB_DIM), jnp.float32),
            pltpu.VMEM((CH // PACK, 128), jnp.float32),
        ],
        compiler_params=pltpu.CompilerParams(needs_layout_passes=False),
    )
    def repack(table_hbm, t2_hbm, in_v, out_v):
        wid = lax.axis_index("s") * NUM_CORES + lax.axis_index("c")
        c_lo = wid * NCHUNK // NUM_WORKERS
        c_hi = (wid + 1) * NCHUNK // NUM_WORKERS

        def chunk(c, _):
            off = pl.multiple_of(c * CH, 8)
            off2 = pl.multiple_of(c * (CH // PACK), 8)
            pltpu.sync_copy(table_hbm.at[pl.ds(off, CH), :], in_v)

            def row(j, _):
                lo = in_v.at[j][pl.ds(0, LANES)]
                hi = in_v.at[j][pl.ds(LANES, LANES)]
                col = (j % PACK) * EMB_DIM
                out_v.at[j // PACK][pl.ds(col, LANES)] = lo
                out_v.at[j // PACK][pl.ds(col + LANES, LANES)] = hi
                return ()

            lax.fori_loop(0, CH, row, (), unroll=8)
            pltpu.sync_copy(out_v, t2_hbm.at[pl.ds(off2, CH // PACK), :])
            return ()

        lax.fori_loop(c_lo, c_hi, chunk, ())

    @functools.partial(
        pl.kernel,
        mesh=mesh,
        out_type=jax.ShapeDtypeStruct((BATCH, 32, 128), jnp.float32),
        scratch_types=[
            pltpu.VMEM((NB, FIELDS), jnp.int32),
            pltpu.VMEM((FIELDS * NB,), jnp.int32),
            pltpu.VMEM((FIELDS * NB,), jnp.int32),
            pltpu.VMEM((2, NG, 128), jnp.float32),
            pltpu.VMEM((NG, 1, EMB_DIM), jnp.float32),
            pltpu.SemaphoreType.DMA((2,)),
        ],
        compiler_params=pltpu.CompilerParams(
            use_tc_tiling_on_sc=False, needs_layout_passes=False
        ),
    )
    def gather(t2_hbm, idx_hbm, out_hbm, idx2d, idx1d, q1d, rows_v, sel_v,
               sem):
        wid = lax.axis_index("s") * NUM_CORES + lax.axis_index("c")
        b0 = wid * NB

        pltpu.sync_copy(idx_hbm.at[pl.ds(b0, NB), :], idx2d)

        lane = lax.iota(jnp.int32, LANES)
        lo_tgt = lane * NB
        hi_tgt = (lane + 10) * NB

        def body(j, _):
            row = idx2d.at[j]
            lo = row[pl.ds(0, LANES)]
            hi = row[pl.ds(10, LANES)]
            plsc.store_scatter(idx1d, [lo_tgt + j], lo)
            plsc.store_scatter(idx1d, [hi_tgt + j], hi)
            plsc.store_scatter(q1d, [lo_tgt + j],
                               lax.shift_right_logical(lo, 2))
            plsc.store_scatter(q1d, [hi_tgt + j],
                               lax.shift_right_logical(hi, 2))
            return ()

        lax.fori_loop(0, NB, body, (), unroll=4)

        NCH = FIELDS * NB // NG

        def fire(g):
            p = lax.rem(g, 2)
            return pltpu.async_copy(
                t2_hbm.at[q1d.at[pl.ds(pl.multiple_of(g * NG, 8), NG)]],
                rows_v.at[p],
                sem.at[p],
            )

        def wait_select_store(g):
            p = lax.rem(g, 2)
            f = g // 2
            half = lax.rem(g, 2)
            pltpu.make_async_copy(
                t2_hbm.at[q1d.at[pl.ds(pl.multiple_of(g * NG, 8), NG)]],
                rows_v.at[p],
                sem.at[p],
            ).wait()

            def sblock(jb, _):
                base = jb * LANES
                v = idx1d[pl.ds(g * NG + base, LANES)]
                cols = lax.bitwise_and(v, 3) * EMB_DIM
                for k in range(LANES):
                    j = base + k
                    col = cols[k]
                    sel_v.at[j, 0][pl.ds(0, LANES)] = (
                        rows_v.at[p, j][pl.ds(col, LANES)]
                    )
                    sel_v.at[j, 0][pl.ds(LANES, LANES)] = (
                        rows_v.at[p, j][pl.ds(col + LANES, LANES)]
                    )
                return ()

            lax.fori_loop(0, NG // LANES, sblock, ())
            pltpu.sync_copy(
                sel_v,
                out_hbm.at[
                    pl.ds(pl.multiple_of(b0 + half * NG, 8), NG),
                    pl.ds(f, 1),
                    pl.ds(0, EMB_DIM),
                ],
            )

        fire(0)

        def pipe(g, _):
            fire(g + 1)
            wait_select_store(g)
            return ()

        lax.fori_loop(0, NCH - 1, pipe, ())
        wait_select_store(NCH - 1)

    t2 = repack(table)
    return gather(t2, d)


def kernel(d, embedding):
    out = _sc_embed(d.astype(jnp.int32), embedding)
    return out[:, :FIELDS, :EMB_DIM]


# speedup vs baseline: 1.4673x; 1.4673x over previous
import functools

import jax
import jax.numpy as jnp
from jax import lax
from jax.experimental import pallas as pl
from jax.experimental.pallas import tpu as pltpu
from jax.experimental.pallas import tpu_sc as plsc

BATCH = 16384
FIELDS = 26
EMB_DIM = 32
LANES = 16

NUM_CORES = 2
NUM_SUBCORES = 16
NUM_WORKERS = NUM_CORES * NUM_SUBCORES
NB = BATCH // NUM_WORKERS


@jax.jit
def _sc_gather(d, table):
    mesh = plsc.VectorSubcoreMesh(core_axis_name="c", subcore_axis_name="s")

    @functools.partial(
        pl.kernel,
        mesh=mesh,
        out_type=jax.ShapeDtypeStruct((BATCH, 32, 128), jnp.float32),
        scratch_types=[
            pltpu.VMEM((NB, FIELDS), jnp.int32),
            pltpu.VMEM((FIELDS * NB,), jnp.int32),
            pltpu.VMEM((2, NB, 1, EMB_DIM), jnp.float32),
            pltpu.SemaphoreType.DMA,
            pltpu.SemaphoreType.DMA,
        ],
        compiler_params=pltpu.CompilerParams(
            use_tc_tiling_on_sc=False, needs_layout_passes=False
        ),
    )
    def k(table_hbm, idx_hbm, out_hbm, idx2d, idx1d, rows_v, sem0, sem1):
        wid = lax.axis_index("s") * NUM_CORES + lax.axis_index("c")
        b0 = wid * NB

        pltpu.sync_copy(idx_hbm.at[pl.ds(b0, NB), :], idx2d)

        lane = lax.iota(jnp.int32, LANES)
        lo_tgt = lane * NB
        hi_tgt = (lane + 10) * NB

        def body(j, _):
            row = idx2d.at[j]
            plsc.store_scatter(idx1d, [lo_tgt + j], row[pl.ds(0, LANES)])
            plsc.store_scatter(idx1d, [hi_tgt + j], row[pl.ds(10, LANES)])
            return ()

        lax.fori_loop(0, NB, body, (), unroll=8)

        sems = (sem0, sem1)
        descs = [None, None]
        for f in range(FIELDS + 1):
            if f < FIELDS:
                p = f % 2
                descs[p] = pltpu.async_copy(
                    table_hbm.at[idx1d.at[pl.ds(f * NB, NB)]],
                    rows_v.at[p, :, 0, :],
                    sems[p],
                )
            if f >= 1:
                q = (f - 1) % 2
                descs[q].wait()
                pltpu.sync_copy(
                    rows_v.at[q],
                    out_hbm.at[
                        pl.ds(b0, NB), pl.ds(f - 1, 1), pl.ds(0, EMB_DIM)
                    ],
                )

    return k(table, d)


def kernel(d, embedding):
    out = _sc_gather(d.astype(jnp.int32), embedding)
    return out[:, :FIELDS, :EMB_DIM]
